# baseline (speedup 1.0000x reference)
import functools

import jax
import jax.numpy as jnp
from jax import lax
from jax.experimental import pallas as pl
from jax.experimental.pallas import tpu as pltpu
from jax.experimental.pallas import tpu_sc as plsc

VOCAB = 100000
EMB = 768
BATCH = 4096
SEQ = 50

NC = 2
NS = 16
NW = NC * NS
BPW = BATCH // NW
LANES = 16
NCH = EMB // LANES
OB = 16
INV_SEQ = 1.0 / SEQ

_mesh = plsc.VectorSubcoreMesh(core_axis_name="c", subcore_axis_name="s")


@functools.partial(
    pl.kernel,
    out_type=jax.ShapeDtypeStruct((BATCH, EMB), jnp.float32),
    mesh=_mesh,
    scratch_types=[
        pltpu.VMEM((BPW, SEQ), jnp.int32),
        pltpu.VMEM((2, 48, EMB), jnp.float32),
        pltpu.VMEM((2, 2, EMB), jnp.float32),
        pltpu.VMEM((2, OB, EMB), jnp.float32),
        pltpu.SemaphoreType.DMA,
        pltpu.SemaphoreType.DMA,
        pltpu.SemaphoreType.DMA,
    ],
)
def _aweencode(ids_hbm, table_hbm, out_hbm, idx_v, rows_v, tail_v, obuf_v,
               gsem0, gsem1, osem):
    wid = lax.axis_index("s") * NC + lax.axis_index("c")
    base = pl.multiple_of(wid * BPW, BPW)
    gsems = (gsem0, gsem1)

    pltpu.sync_copy(ids_hbm.at[pl.ds(base, BPW)], idx_v)

    def gather_starts(r, b):
        return (
            pltpu.make_async_copy(
                table_hbm.at[idx_v.at[r, pl.ds(0, 24)]],
                rows_v.at[b, pl.ds(0, 24)], gsems[b]),
            pltpu.make_async_copy(
                table_hbm.at[idx_v.at[r, pl.ds(24, 24)]],
                rows_v.at[b, pl.ds(24, 24)], gsems[b]),
            pltpu.make_async_copy(
                table_hbm.at[idx_v.at[r, pl.ds(48, 2)]],
                tail_v.at[b], gsems[b]),
        )

    def gather_start(r, b):
        for g in gather_starts(r, b):
            g.start()

    def gather_wait(r, b):
        for g in gather_starts(r, b):
            g.wait()

    gather_start(0, 0)
    gather_start(1, 1)

    def row_body(g, _):
        for b in range(2):
            r = 2 * g + b
            gather_wait(r, b)
            slot = lax.rem(r, OB)

            HALF = NCH // 2
            for h in range(2):
                chunks = range(h * HALF, (h + 1) * HALF)

                def sl(c):
                    return pl.ds(c * LANES, LANES)

                acc = tuple(rows_v[b, 0, sl(c)] + tail_v[b, 0, sl(c)] +
                            tail_v[b, 1, sl(c)] for c in chunks)

                def add_row(j, acc):
                    return tuple(a + rows_v[b, j, sl(c)]
                                 for a, c in zip(acc, chunks))

                acc = lax.fori_loop(1, 48, add_row, acc, unroll=2)
                osel = lax.rem(lax.div(r, OB), 2)
                for a, c in zip(acc, chunks):
                    obuf_v[osel, slot, sl(c)] = a * INV_SEQ

            @pl.when(r < BPW - 2)
            def _():
                gather_start(r + 2, b)

            def flush(r):
                osel = lax.rem(lax.div(r, OB), 2)
                off = pl.multiple_of(base + r - (OB - 1), OB)
                return pltpu.make_async_copy(
                    obuf_v.at[osel], out_hbm.at[pl.ds(off, OB)], osem)

            @pl.when(slot == OB - 1)
            def _():
                @pl.when(r >= 2 * OB)
                def _():
                    flush(r - OB).wait()
                flush(r).start()
        return ()

    lax.fori_loop(0, BPW // 2, row_body, (), unroll=False)
    flush_last = pltpu.make_async_copy(
        obuf_v.at[lax.rem(lax.div(BPW - 1, OB), 2)],
        out_hbm.at[pl.ds(base + BPW - OB, OB)], osem)
    flush_prev = pltpu.make_async_copy(
        obuf_v.at[lax.rem(lax.div(BPW - 1 - OB, OB), 2)],
        out_hbm.at[pl.ds(base + BPW - 2 * OB, OB)], osem)
    flush_prev.wait()
    flush_last.wait()


def kernel(input_ids, table):
    return _aweencode(input_ids.astype(jnp.int32), table)


# speedup vs baseline: 1.0017x; 1.0017x over previous
import functools

import jax
import jax.numpy as jnp
from jax import lax
from jax.experimental import pallas as pl
from jax.experimental.pallas import tpu as pltpu
from jax.experimental.pallas import tpu_sc as plsc

VOCAB = 100000
EMB = 768
BATCH = 4096
SEQ = 50

NC = 2
NS = 16
NW = NC * NS
BPW = BATCH // NW
LANES = 16
NCH = EMB // LANES
OB = 16
INV_SEQ = 1.0 / SEQ

_mesh = plsc.VectorSubcoreMesh(core_axis_name="c", subcore_axis_name="s")


@functools.partial(
    pl.kernel,
    out_type=jax.ShapeDtypeStruct((BATCH, EMB), jnp.float32),
    mesh=_mesh,
    scratch_types=[
        pltpu.VMEM((BPW, SEQ), jnp.int32),
        pltpu.VMEM((2, 48, EMB), jnp.float32),
        pltpu.VMEM((2, 2, EMB), jnp.float32),
        pltpu.VMEM((2, OB, EMB), jnp.float32),
        pltpu.SemaphoreType.DMA,
        pltpu.SemaphoreType.DMA,
        pltpu.SemaphoreType.DMA,
    ],
)
def _aweencode(ids_hbm, table_hbm, out_hbm, idx_v, rows_v, tail_v, obuf_v,
               gsem0, gsem1, osem):
    wid = lax.axis_index("s") * NC + lax.axis_index("c")
    base = pl.multiple_of(wid * BPW, BPW)
    gsems = (gsem0, gsem1)

    pltpu.sync_copy(ids_hbm.at[pl.ds(base, BPW)], idx_v)

    def gather_starts(r, b):
        return (
            pltpu.make_async_copy(
                table_hbm.at[idx_v.at[r, pl.ds(0, 48)]],
                rows_v.at[b], gsems[b]),
            pltpu.make_async_copy(
                table_hbm.at[idx_v.at[r, pl.ds(48, 2)]],
                tail_v.at[b], gsems[b]),
        )

    def gather_start(r, b):
        for g in gather_starts(r, b):
            g.start()

    def gather_wait(r, b):
        for g in gather_starts(r, b):
            g.wait()

    gather_start(0, 0)
    gather_start(1, 1)

    def row_body(g, _):
        for b in range(2):
            r = 2 * g + b
            gather_wait(r, b)
            slot = lax.rem(r, OB)

            HALF = NCH // 2
            for h in range(2):
                chunks = range(h * HALF, (h + 1) * HALF)

                def sl(c):
                    return pl.ds(c * LANES, LANES)

                acc = tuple(rows_v[b, 0, sl(c)] + tail_v[b, 0, sl(c)] +
                            tail_v[b, 1, sl(c)] for c in chunks)

                def add_row(j, acc):
                    return tuple(a + rows_v[b, j, sl(c)]
                                 for a, c in zip(acc, chunks))

                acc = lax.fori_loop(1, 48, add_row, acc, unroll=2)
                osel = lax.rem(lax.div(r, OB), 2)
                for a, c in zip(acc, chunks):
                    obuf_v[osel, slot, sl(c)] = a * INV_SEQ

            @pl.when(r < BPW - 2)
            def _():
                gather_start(r + 2, b)

            def flush(r):
                osel = lax.rem(lax.div(r, OB), 2)
                off = pl.multiple_of(base + r - (OB - 1), OB)
                return pltpu.make_async_copy(
                    obuf_v.at[osel], out_hbm.at[pl.ds(off, OB)], osem)

            @pl.when(slot == OB - 1)
            def _():
                @pl.when(r >= 2 * OB)
                def _():
                    flush(r - OB).wait()
                flush(r).start()
        return ()

    lax.fori_loop(0, BPW // 2, row_body, (), unroll=False)
    flush_last = pltpu.make_async_copy(
        obuf_v.at[lax.rem(lax.div(BPW - 1, OB), 2)],
        out_hbm.at[pl.ds(base + BPW - OB, OB)], osem)
    flush_prev = pltpu.make_async_copy(
        obuf_v.at[lax.rem(lax.div(BPW - 1 - OB, OB), 2)],
        out_hbm.at[pl.ds(base + BPW - 2 * OB, OB)], osem)
    flush_prev.wait()
    flush_last.wait()


def kernel(input_ids, table):
    return _aweencode(input_ids.astype(jnp.int32), table)


# speedup vs baseline: 1.0036x; 1.0019x over previous
import functools

import jax
import jax.numpy as jnp
from jax import lax
from jax.experimental import pallas as pl
from jax.experimental.pallas import tpu as pltpu
from jax.experimental.pallas import tpu_sc as plsc

VOCAB = 100000
EMB = 768
BATCH = 4096
SEQ = 50

NC = 2
NS = 16
NW = NC * NS
BPW = BATCH // NW
LANES = 16
NCH = EMB // LANES
OB = 16
INV_SEQ = 1.0 / SEQ

_mesh = plsc.VectorSubcoreMesh(core_axis_name="c", subcore_axis_name="s")


@functools.partial(
    pl.kernel,
    out_type=jax.ShapeDtypeStruct((BATCH, EMB), jnp.float32),
    mesh=_mesh,
    scratch_types=[
        pltpu.VMEM((BPW, SEQ), jnp.int32),
        pltpu.VMEM((2, 48, EMB), jnp.float32),
        pltpu.VMEM((2, 2, EMB), jnp.float32),
        pltpu.VMEM((2, OB, EMB), jnp.float32),
        pltpu.SemaphoreType.DMA,
        pltpu.SemaphoreType.DMA,
        pltpu.SemaphoreType.DMA,
    ],
)
def _aweencode(ids_hbm, table_hbm, out_hbm, idx_v, rows_v, tail_v, obuf_v,
               gsem0, gsem1, osem):
    wid = lax.axis_index("s") * NC + lax.axis_index("c")
    base = pl.multiple_of(wid * BPW, BPW)
    gsems = (gsem0, gsem1)

    pltpu.sync_copy(ids_hbm.at[pl.ds(base, BPW)], idx_v)

    def gather_starts(r, b):
        return (
            pltpu.make_async_copy(
                table_hbm.at[idx_v.at[r, pl.ds(0, 48)]],
                rows_v.at[b], gsems[b]),
            pltpu.make_async_copy(
                table_hbm.at[idx_v.at[r, pl.ds(48, 2)]],
                tail_v.at[b], gsems[b]),
        )

    def gather_start(r, b):
        for g in gather_starts(r, b):
            g.start()

    def gather_wait(r, b):
        for g in gather_starts(r, b):
            g.wait()

    def flush(r):
        osel = lax.rem(lax.div(r, OB), 2)
        off = pl.multiple_of(base + r - (OB - 1), OB)
        return pltpu.make_async_copy(
            obuf_v.at[osel], out_hbm.at[pl.ds(off, OB)], osem)

    gather_start(0, 0)
    gather_start(1, 1)

    def row_body(g, _):
        for b in range(2):
            r = 2 * g + b
            gather_wait(r, b)
            slot = lax.rem(r, OB)

            @pl.when(jnp.logical_and(slot == 0, r >= 2 * OB))
            def _():
                flush(r - OB - 1).wait()

            HALF = NCH // 2
            for h in range(2):
                chunks = range(h * HALF, (h + 1) * HALF)

                def sl(c):
                    return pl.ds(c * LANES, LANES)

                acc = tuple(rows_v[b, 0, sl(c)] + tail_v[b, 0, sl(c)] +
                            tail_v[b, 1, sl(c)] for c in chunks)

                def add_row(j, acc):
                    return tuple(a + rows_v[b, j, sl(c)]
                                 for a, c in zip(acc, chunks))

                acc = lax.fori_loop(1, 48, add_row, acc, unroll=2)
                osel = lax.rem(lax.div(r, OB), 2)
                for a, c in zip(acc, chunks):
                    obuf_v[osel, slot, sl(c)] = a * INV_SEQ

            @pl.when(r < BPW - 2)
            def _():
                gather_start(r + 2, b)

            @pl.when(slot == OB - 1)
            def _():
                flush(r).start()
        return ()

    lax.fori_loop(0, BPW // 2, row_body, (), unroll=False)
    flush(BPW - 1 - OB).wait()
    flush(BPW - 1).wait()


def kernel(input_ids, table):
    return _aweencode(input_ids.astype(jnp.int32), table)
